# Initial kernel scaffold; baseline (speedup 1.0000x reference)
import jax
import jax.numpy as jnp
from jax.experimental import pallas as pl


def kernel(x, edge_index, W_self1, W_neigh1, b1, W_self2, W_neigh2, b2):
    raise NotImplementedError("write your pallas kernel here")


# speedup vs baseline: 4.8647x; 4.8647x over previous
import functools

import jax
import jax.numpy as jnp
from jax import lax
from jax.experimental import pallas as pl
from jax.experimental.pallas import tpu as pltpu
from jax.experimental.pallas import tpu_sc as plsc

N = 10000
E = 320000
D = 128
NC = 2
NS = 16
NP = 10240
RPT = NP // NS

EC = E // NC
ET = EC // NS
FCHUNKS, FTAIL = ET // 128, ET % 128
DT = E // NS
DCHUNKS, DTAIL = DT // 128, DT % 128
GT = E // (NC * NS)
GCHUNKS, GTAIL = GT // 128, GT % 128

_mesh = plsc.VectorSubcoreMesh(core_axis_name="c", subcore_axis_name="s")


def _fill_ones(ref, n):
    @pl.loop(0, n // 16)
    def _(i):
        ref[pl.ds(i * 16, 16)] = jnp.ones((16,), jnp.float32)


def _scale_and_writeback(agg_sh, inv_v, rows_buf, part_hbm, r0, core_row0):
    @pl.loop(0, RPT // 64)
    def _(j):
        rbase = r0 + j * 64
        pltpu.sync_copy(agg_sh.at[pl.ds(rbase, 64)], rows_buf)

        @pl.loop(0, 4)
        def _(rr):
            ivec = inv_v[pl.ds(j * 64 + rr * 16, 16)]
            for r in range(16):
                sc = ivec[r]
                row = rr * 16 + r
                for k in range(8):
                    rows_buf[row, pl.ds(k * 16, 16)] = (
                        rows_buf[row, pl.ds(k * 16, 16)] * sc)

        pltpu.sync_copy(rows_buf, part_hbm.at[pl.ds(core_row0 + rbase, 64)])


def _agg_phase(feat_hbm, src_hbm, dst_hbm, agg_sh, sidx, didx, sidx_t, didx_t,
               rows_v, rows_t, sem, f_base):
    @pl.loop(0, FCHUNKS)
    def _(j):
        off = pl.multiple_of(f_base + j * 128, 8)
        pltpu.sync_copy(src_hbm.at[pl.ds(off, 128)], sidx)
        pltpu.sync_copy(dst_hbm.at[pl.ds(off, 128)], didx)
        pltpu.async_copy(feat_hbm.at[sidx], rows_v, sem).wait()
        pltpu.sync_copy(rows_v, agg_sh.at[didx], add=True)

    off = f_base + FCHUNKS * 128
    pltpu.sync_copy(src_hbm.at[pl.ds(off, FTAIL)], sidx_t)
    pltpu.sync_copy(dst_hbm.at[pl.ds(off, FTAIL)], didx_t)
    pltpu.async_copy(feat_hbm.at[sidx_t], rows_t, sem).wait()
    pltpu.sync_copy(rows_t, agg_sh.at[didx_t], add=True)


def _sc_agg_layer1(x, src, dst, z2, z1):
    @functools.partial(
        pl.kernel,
        out_type=(
            jax.ShapeDtypeStruct((2 * NP, D), jnp.float32),
            jax.ShapeDtypeStruct((NP,), jnp.float32),
        ),
        mesh=_mesh,
        scratch_types=dict(
            agg_sh=pltpu.VMEM_SHARED((NP, D), jnp.float32),
            deg_sh=pltpu.VMEM_SHARED((NP,), jnp.float32),
            sidx=pltpu.VMEM((128,), jnp.int32),
            didx=pltpu.VMEM((128,), jnp.int32),
            sidx_t=pltpu.VMEM((FTAIL,), jnp.int32),
            didx_t=pltpu.VMEM((FTAIL,), jnp.int32),
            didx_d=pltpu.VMEM((128,), jnp.int32),
            didx_dt=pltpu.VMEM((DTAIL,), jnp.int32),
            ones_v=pltpu.VMEM((128,), jnp.float32),
            ones_t=pltpu.VMEM((DTAIL,), jnp.float32),
            rows_v=pltpu.VMEM((128, D), jnp.float32),
            rows_t=pltpu.VMEM((FTAIL, D), jnp.float32),
            deg_v=pltpu.VMEM((RPT,), jnp.float32),
            inv_v=pltpu.VMEM((RPT,), jnp.float32),
            rows_buf=pltpu.VMEM((64, D), jnp.float32),
            sem=pltpu.SemaphoreType.DMA,
        ),
    )
    def k(x_hbm, src_hbm, dst_hbm, z2_hbm, z1_hbm, part_hbm, inv_hbm, *,
          agg_sh, deg_sh, sidx, didx, sidx_t, didx_t, didx_d, didx_dt,
          ones_v, ones_t, rows_v, rows_t, deg_v, inv_v, rows_buf, sem):
        c = lax.axis_index("c")
        s = lax.axis_index("s")
        r0 = s * RPT

        pltpu.sync_copy(z2_hbm.at[pl.ds(r0, RPT)], agg_sh.at[pl.ds(r0, RPT)])
        pltpu.sync_copy(z1_hbm.at[pl.ds(r0, RPT)], deg_sh.at[pl.ds(r0, RPT)])
        _fill_ones(ones_v, 128)
        _fill_ones(ones_t, DTAIL)
        plsc.subcore_barrier()

        d_base = s * DT

        @pl.loop(0, DCHUNKS)
        def _(j):
            off = pl.multiple_of(d_base + j * 128, 8)
            pltpu.sync_copy(dst_hbm.at[pl.ds(off, 128)], didx_d)
            pltpu.sync_copy(ones_v, deg_sh.at[didx_d], add=True)

        offd = d_base + DCHUNKS * 128
        pltpu.sync_copy(dst_hbm.at[pl.ds(offd, DTAIL)], didx_dt)
        pltpu.sync_copy(ones_t, deg_sh.at[didx_dt], add=True)

        f_base = c * EC + s * ET
        _agg_phase(x_hbm, src_hbm, dst_hbm, agg_sh, sidx, didx, sidx_t,
                   didx_t, rows_v, rows_t, sem, f_base)
        plsc.subcore_barrier()

        pltpu.sync_copy(deg_sh.at[pl.ds(r0, RPT)], deg_v)

        @pl.loop(0, RPT // 16)
        def _(i):
            dchunk = deg_v[pl.ds(i * 16, 16)]
            inv_v[pl.ds(i * 16, 16)] = 1.0 / jnp.maximum(dchunk, 1.0)

        @pl.when(c == 0)
        def _():
            pltpu.sync_copy(inv_v, inv_hbm.at[pl.ds(r0, RPT)])

        _scale_and_writeback(agg_sh, inv_v, rows_buf, part_hbm, r0, c * NP)

    return k(x, src, dst, z2, z1)


def _sc_agg_layer2(h, src, dst, inv, z2):
    @functools.partial(
        pl.kernel,
        out_type=jax.ShapeDtypeStruct((2 * NP, D), jnp.float32),
        mesh=_mesh,
        scratch_types=dict(
            agg_sh=pltpu.VMEM_SHARED((NP, D), jnp.float32),
            sidx=pltpu.VMEM((128,), jnp.int32),
            didx=pltpu.VMEM((128,), jnp.int32),
            sidx_t=pltpu.VMEM((FTAIL,), jnp.int32),
            didx_t=pltpu.VMEM((FTAIL,), jnp.int32),
            rows_v=pltpu.VMEM((128, D), jnp.float32),
            rows_t=pltpu.VMEM((FTAIL, D), jnp.float32),
            inv_v=pltpu.VMEM((RPT,), jnp.float32),
            rows_buf=pltpu.VMEM((64, D), jnp.float32),
            sem=pltpu.SemaphoreType.DMA,
        ),
    )
    def k(h_hbm, src_hbm, dst_hbm, inv_hbm, z2_hbm, part_hbm, *,
          agg_sh, sidx, didx, sidx_t, didx_t, rows_v, rows_t, inv_v,
          rows_buf, sem):
        c = lax.axis_index("c")
        s = lax.axis_index("s")
        r0 = s * RPT

        pltpu.sync_copy(z2_hbm.at[pl.ds(r0, RPT)], agg_sh.at[pl.ds(r0, RPT)])
        plsc.subcore_barrier()

        f_base = c * EC + s * ET
        _agg_phase(h_hbm, src_hbm, dst_hbm, agg_sh, sidx, didx, sidx_t,
                   didx_t, rows_v, rows_t, sem, f_base)
        plsc.subcore_barrier()

        pltpu.sync_copy(inv_hbm.at[pl.ds(r0, RPT)], inv_v)
        _scale_and_writeback(agg_sh, inv_v, rows_buf, part_hbm, r0, c * NP)

    return k(h, src, dst, inv, z2)


def _sc_gather_out(y, src, dst):
    @functools.partial(
        pl.kernel,
        out_type=(
            jax.ShapeDtypeStruct((E, D), jnp.float32),
            jax.ShapeDtypeStruct((E, D), jnp.float32),
        ),
        mesh=_mesh,
        scratch_types=dict(
            sidx=pltpu.VMEM((128,), jnp.int32),
            didx=pltpu.VMEM((128,), jnp.int32),
            sidx_t=pltpu.VMEM((GTAIL,), jnp.int32),
            didx_t=pltpu.VMEM((GTAIL,), jnp.int32),
            rows_a=pltpu.VMEM((128, D), jnp.float32),
            rows_b=pltpu.VMEM((128, D), jnp.float32),
            rows_ta=pltpu.VMEM((GTAIL, D), jnp.float32),
            rows_tb=pltpu.VMEM((GTAIL, D), jnp.float32),
            sem_a=pltpu.SemaphoreType.DMA,
            sem_b=pltpu.SemaphoreType.DMA,
        ),
    )
    def k(y_hbm, src_hbm, dst_hbm, sf_hbm, df_hbm, *,
          sidx, didx, sidx_t, didx_t, rows_a, rows_b, rows_ta, rows_tb,
          sem_a, sem_b):
        c = lax.axis_index("c")
        s = lax.axis_index("s")
        base = (c * NS + s) * GT

        @pl.loop(0, GCHUNKS)
        def _(j):
            off = pl.multiple_of(base + j * 128, 8)
            pltpu.sync_copy(src_hbm.at[pl.ds(off, 128)], sidx)
            pltpu.sync_copy(dst_hbm.at[pl.ds(off, 128)], didx)
            ca = pltpu.async_copy(y_hbm.at[sidx], rows_a, sem_a)
            cb = pltpu.async_copy(y_hbm.at[didx], rows_b, sem_b)
            ca.wait()
            pltpu.sync_copy(rows_a, sf_hbm.at[pl.ds(off, 128)])
            cb.wait()
            pltpu.sync_copy(rows_b, df_hbm.at[pl.ds(off, 128)])

        off = base + GCHUNKS * 128
        pltpu.sync_copy(src_hbm.at[pl.ds(off, GTAIL)], sidx_t)
        pltpu.sync_copy(dst_hbm.at[pl.ds(off, GTAIL)], didx_t)
        ca = pltpu.async_copy(y_hbm.at[sidx_t], rows_ta, sem_a)
        cb = pltpu.async_copy(y_hbm.at[didx_t], rows_tb, sem_b)
        ca.wait()
        pltpu.sync_copy(rows_ta, sf_hbm.at[pl.ds(off, GTAIL)])
        cb.wait()
        pltpu.sync_copy(rows_tb, df_hbm.at[pl.ds(off, GTAIL)])

    return k(y, src, dst)


def _tc_dense(x, part, W_s, W_n, b, relu):
    R = 1000
    part3 = part.reshape(2, NP, D)
    b2d = b.reshape(1, D)

    def body(x_ref, p0_ref, p1_ref, ws_ref, wn_ref, b_ref, o_ref):
        acc = jnp.dot(x_ref[...], ws_ref[...], preferred_element_type=jnp.float32)
        acc = acc + jnp.dot(p0_ref[0] + p1_ref[0], wn_ref[...],
                            preferred_element_type=jnp.float32)
        acc = acc + b_ref[...]
        if relu:
            acc = jnp.maximum(acc, 0.0)
        o_ref[...] = acc

    return pl.pallas_call(
        body,
        grid=(N // R,),
        in_specs=[
            pl.BlockSpec((R, D), lambda i: (i, 0)),
            pl.BlockSpec((1, R, D), lambda i: (0, i, 0)),
            pl.BlockSpec((1, R, D), lambda i: (1, i, 0)),
            pl.BlockSpec((D, D), lambda i: (0, 0)),
            pl.BlockSpec((D, D), lambda i: (0, 0)),
            pl.BlockSpec((1, D), lambda i: (0, 0)),
        ],
        out_specs=pl.BlockSpec((R, D), lambda i: (i, 0)),
        out_shape=jax.ShapeDtypeStruct((N, D), jnp.float32),
    )(x, part3, part3, W_s, W_n, b2d)


def kernel(x, edge_index, W_self1, W_neigh1, b1, W_self2, W_neigh2, b2):
    src = edge_index[0].astype(jnp.int32)
    dst = edge_index[1].astype(jnp.int32)
    z2 = jnp.zeros((NP, D), jnp.float32)
    z1 = jnp.zeros((NP,), jnp.float32)

    part1, inv = _sc_agg_layer1(x, src, dst, z2, z1)
    h = _tc_dense(x, part1, W_self1, W_neigh1, b1, relu=True)
    part2 = _sc_agg_layer2(h, src, dst, inv, z2)
    out2 = _tc_dense(h, part2, W_self2, W_neigh2, b2, relu=False)
    src_feat, dst_feat = _sc_gather_out(out2, src, dst)
    return (src_feat, dst_feat)


# speedup vs baseline: 8.9489x; 1.8396x over previous
import functools

import jax
import jax.numpy as jnp
from jax import lax
from jax.experimental import pallas as pl
from jax.experimental.pallas import tpu as pltpu
from jax.experimental.pallas import tpu_sc as plsc

N = 10000
E = 320000
D = 128
NC = 2
NS = 16
NP = 10240
RPT = NP // NS

EC = E // NC
ET = EC // NS
FCH, FTAIL = ET // 128, ET % 128
DT = E // NS
DCH, DTAIL = DT // 128, DT % 128
GT = E // (NC * NS)
GCH, GTAIL = GT // 128, GT % 128

_mesh = plsc.VectorSubcoreMesh(core_axis_name="c", subcore_axis_name="s")

_IDX4 = [pltpu.VMEM((128,), jnp.int32) for _ in range(4)]
_SEM4 = [pltpu.SemaphoreType.DMA for _ in range(4)]
_SEM2 = [pltpu.SemaphoreType.DMA for _ in range(2)]


def _fill_ones(ref, n):
    @pl.loop(0, n // 16)
    def _(i):
        ref[pl.ds(i * 16, 16)] = jnp.ones((16,), jnp.float32)


def _scale_and_writeback(agg_sh, inv_v, rows_buf, part_hbm, r0, core_row0):
    @pl.loop(0, RPT // 64)
    def _(j):
        rbase = r0 + j * 64
        pltpu.sync_copy(agg_sh.at[pl.ds(rbase, 64)], rows_buf)

        @pl.loop(0, 4)
        def _(rr):
            ivec = inv_v[pl.ds(j * 64 + rr * 16, 16)]
            for r in range(16):
                sc = ivec[r]
                row = rr * 16 + r
                for k in range(8):
                    rows_buf[row, pl.ds(k * 16, 16)] = (
                        rows_buf[row, pl.ds(k * 16, 16)] * sc)

        pltpu.sync_copy(rows_buf, part_hbm.at[pl.ds(core_row0 + rbase, 64)])


def _pipelined_agg(feat_hbm, src_hbm, dst_hbm, agg_sh, sidx, didx, rows,
                   isem, gsem, ssem, sidx_t, didx_t, rows_t, f_base):
    def idx_issue(jj, b4):
        off = pl.multiple_of(f_base + jj * 128, 8)
        pltpu.async_copy(src_hbm.at[pl.ds(off, 128)], sidx[b4], isem[b4])
        pltpu.async_copy(dst_hbm.at[pl.ds(off, 128)], didx[b4], isem[b4])

    def idx_wait(b4):
        pltpu.make_async_copy(src_hbm.at[pl.ds(0, 128)], sidx[b4], isem[b4]).wait()
        pltpu.make_async_copy(dst_hbm.at[pl.ds(0, 128)], didx[b4], isem[b4]).wait()

    def gather_issue(b4, b2):
        pltpu.async_copy(feat_hbm.at[sidx[b4]], rows[b2], gsem[b2])

    def gather_wait(b4, b2):
        pltpu.make_async_copy(feat_hbm.at[sidx[b4]], rows[b2], gsem[b2]).wait()

    def scat_issue(b4, b2):
        pltpu.async_copy(rows[b2], agg_sh.at[didx[b4]], ssem[b2], add=True)

    def scat_wait(b4, b2):
        pltpu.make_async_copy(rows[b2], agg_sh.at[didx[b4]], ssem[b2]).wait()

    def B(jj, u, issue_idx=True, first=False, second=False):
        b2 = u % 2
        b4 = (2 + u) % 4 if not (first or second) else (0 if first else 1)
        if not first:
            pass
        if not (first or second):
            scat_wait(u % 4, b2)
        if issue_idx:
            idx_issue(jj + 2, u % 4 if not (first or second) else (2 if first else 3))
        idx_wait(b4)
        gather_issue(b4, b2)
        if not first:
            b4p = (b4 + 3) % 4
            gather_wait(b4p, 1 - b2)
            scat_issue(b4p, 1 - b2)

    idx_issue(0, 0)
    idx_issue(1, 1)
    B(0, 0, first=True)
    B(1, 1, second=True)

    @pl.loop(2, FCH - 4, step=4)
    def _(v):
        for u in range(4):
            B(v + u, u)

    B(FCH - 4, 0)
    B(FCH - 3, 1)
    B(FCH - 2, 2, issue_idx=False)
    B(FCH - 1, 3, issue_idx=False)

    gather_wait((FCH - 1) % 4, 1)
    scat_issue((FCH - 1) % 4, 1)
    scat_wait((FCH - 2) % 4, 0)
    scat_wait((FCH - 1) % 4, 1)

    off = f_base + FCH * 128
    pltpu.sync_copy(src_hbm.at[pl.ds(off, FTAIL)], sidx_t)
    pltpu.sync_copy(dst_hbm.at[pl.ds(off, FTAIL)], didx_t)
    pltpu.async_copy(feat_hbm.at[sidx_t], rows_t, gsem[0]).wait()
    pltpu.sync_copy(rows_t, agg_sh.at[didx_t], add=True)


def _pipelined_deg(dst_hbm, deg_sh, didx_d, ones_v, ones_t, didx_dt,
                   di, ds, d_base):
    def idx_issue(jj, b4):
        off = pl.multiple_of(d_base + jj * 128, 8)
        pltpu.async_copy(dst_hbm.at[pl.ds(off, 128)], didx_d[b4], di[b4])

    def idx_wait(b4):
        pltpu.make_async_copy(dst_hbm.at[pl.ds(0, 128)], didx_d[b4], di[b4]).wait()

    def scat_issue(b4, b2):
        pltpu.async_copy(ones_v, deg_sh.at[didx_d[b4]], ds[b2], add=True)

    def scat_wait(b4, b2):
        pltpu.make_async_copy(ones_v, deg_sh.at[didx_d[b4]], ds[b2]).wait()

    def DD(jj, b4, issue_idx=True, warm=False):
        b2 = b4 % 2
        if not warm:
            scat_wait((b4 + 2) % 4, b2)
        if issue_idx:
            idx_issue(jj + 2, (b4 + 2) % 4)
        idx_wait(b4)
        scat_issue(b4, b2)

    idx_issue(0, 0)
    idx_issue(1, 1)
    DD(0, 0, warm=True)
    DD(1, 1, warm=True)

    @pl.loop(2, DCH - 2, step=4)
    def _(v):
        for u in range(4):
            DD(v + u, (2 + u) % 4)

    DD(DCH - 2, 2, issue_idx=False)
    DD(DCH - 1, 3, issue_idx=False)
    scat_wait(2, 0)
    scat_wait(3, 1)

    off = d_base + DCH * 128
    pltpu.sync_copy(dst_hbm.at[pl.ds(off, DTAIL)], didx_dt)
    pltpu.sync_copy(ones_t, deg_sh.at[didx_dt], add=True)


def _sc_agg_layer1(x, src, dst, z2, z1):
    @functools.partial(
        pl.kernel,
        out_type=(
            jax.ShapeDtypeStruct((2 * NP, D), jnp.float32),
            jax.ShapeDtypeStruct((NP,), jnp.float32),
        ),
        mesh=_mesh,
        scratch_types=dict(
            agg_sh=pltpu.VMEM_SHARED((NP, D), jnp.float32),
            deg_sh=pltpu.VMEM_SHARED((NP,), jnp.float32),
            sidx=list(_IDX4), didx=list(_IDX4), didx_d=list(_IDX4),
            sidx_t=pltpu.VMEM((FTAIL,), jnp.int32),
            didx_t=pltpu.VMEM((FTAIL,), jnp.int32),
            didx_dt=pltpu.VMEM((DTAIL,), jnp.int32),
            ones_v=pltpu.VMEM((128,), jnp.float32),
            ones_t=pltpu.VMEM((DTAIL,), jnp.float32),
            rows=[pltpu.VMEM((128, D), jnp.float32) for _ in range(2)],
            rows_t=pltpu.VMEM((FTAIL, D), jnp.float32),
            deg_v=pltpu.VMEM((RPT,), jnp.float32),
            inv_v=pltpu.VMEM((RPT,), jnp.float32),
            rows_buf=pltpu.VMEM((64, D), jnp.float32),
            isem=list(_SEM4), gsem=list(_SEM2), ssem=list(_SEM2),
            di=list(_SEM4), ds=list(_SEM2),
        ),
    )
    def k(x_hbm, src_hbm, dst_hbm, z2_hbm, z1_hbm, part_hbm, inv_hbm, *,
          agg_sh, deg_sh, sidx, didx, didx_d, sidx_t, didx_t, didx_dt,
          ones_v, ones_t, rows, rows_t, deg_v, inv_v, rows_buf,
          isem, gsem, ssem, di, ds):
        c = lax.axis_index("c")
        s = lax.axis_index("s")
        r0 = s * RPT

        pltpu.sync_copy(z2_hbm.at[pl.ds(r0, RPT)], agg_sh.at[pl.ds(r0, RPT)])
        pltpu.sync_copy(z1_hbm.at[pl.ds(r0, RPT)], deg_sh.at[pl.ds(r0, RPT)])
        _fill_ones(ones_v, 128)
        _fill_ones(ones_t, DTAIL)
        plsc.subcore_barrier()

        _pipelined_deg(dst_hbm, deg_sh, didx_d, ones_v, ones_t, didx_dt,
                       di, ds, s * DT)

        _pipelined_agg(x_hbm, src_hbm, dst_hbm, agg_sh, sidx, didx, rows,
                       isem, gsem, ssem, sidx_t, didx_t, rows_t,
                       c * EC + s * ET)
        plsc.subcore_barrier()

        pltpu.sync_copy(deg_sh.at[pl.ds(r0, RPT)], deg_v)

        @pl.loop(0, RPT // 16)
        def _(i):
            dchunk = deg_v[pl.ds(i * 16, 16)]
            inv_v[pl.ds(i * 16, 16)] = 1.0 / jnp.maximum(dchunk, 1.0)

        @pl.when(c == 0)
        def _():
            pltpu.sync_copy(inv_v, inv_hbm.at[pl.ds(r0, RPT)])

        _scale_and_writeback(agg_sh, inv_v, rows_buf, part_hbm, r0, c * NP)

    return k(x, src, dst, z2, z1)


def _sc_agg_layer2(h, src, dst, inv, z2):
    @functools.partial(
        pl.kernel,
        out_type=jax.ShapeDtypeStruct((2 * NP, D), jnp.float32),
        mesh=_mesh,
        scratch_types=dict(
            agg_sh=pltpu.VMEM_SHARED((NP, D), jnp.float32),
            sidx=list(_IDX4), didx=list(_IDX4),
            sidx_t=pltpu.VMEM((FTAIL,), jnp.int32),
            didx_t=pltpu.VMEM((FTAIL,), jnp.int32),
            rows=[pltpu.VMEM((128, D), jnp.float32) for _ in range(2)],
            rows_t=pltpu.VMEM((FTAIL, D), jnp.float32),
            inv_v=pltpu.VMEM((RPT,), jnp.float32),
            rows_buf=pltpu.VMEM((64, D), jnp.float32),
            isem=list(_SEM4), gsem=list(_SEM2), ssem=list(_SEM2),
        ),
    )
    def k(h_hbm, src_hbm, dst_hbm, inv_hbm, z2_hbm, part_hbm, *,
          agg_sh, sidx, didx, sidx_t, didx_t, rows, rows_t, inv_v,
          rows_buf, isem, gsem, ssem):
        c = lax.axis_index("c")
        s = lax.axis_index("s")
        r0 = s * RPT

        pltpu.sync_copy(z2_hbm.at[pl.ds(r0, RPT)], agg_sh.at[pl.ds(r0, RPT)])
        plsc.subcore_barrier()

        _pipelined_agg(h_hbm, src_hbm, dst_hbm, agg_sh, sidx, didx, rows,
                       isem, gsem, ssem, sidx_t, didx_t, rows_t,
                       c * EC + s * ET)
        plsc.subcore_barrier()

        pltpu.sync_copy(inv_hbm.at[pl.ds(r0, RPT)], inv_v)
        _scale_and_writeback(agg_sh, inv_v, rows_buf, part_hbm, r0, c * NP)

    return k(h, src, dst, inv, z2)


def _sc_gather_out(y, src, dst):
    @functools.partial(
        pl.kernel,
        out_type=(
            jax.ShapeDtypeStruct((E, D), jnp.float32),
            jax.ShapeDtypeStruct((E, D), jnp.float32),
        ),
        mesh=_mesh,
        scratch_types=dict(
            sidx=list(_IDX4), didx=list(_IDX4),
            sidx_t=pltpu.VMEM((GTAIL,), jnp.int32),
            didx_t=pltpu.VMEM((GTAIL,), jnp.int32),
            rows_a=[pltpu.VMEM((128, D), jnp.float32) for _ in range(2)],
            rows_b=[pltpu.VMEM((128, D), jnp.float32) for _ in range(2)],
            rows_ta=pltpu.VMEM((GTAIL, D), jnp.float32),
            rows_tb=pltpu.VMEM((GTAIL, D), jnp.float32),
            isem=list(_SEM4), ga=list(_SEM2), gb=list(_SEM2),
            wa=list(_SEM2), wb=list(_SEM2),
        ),
    )
    def k(y_hbm, src_hbm, dst_hbm, sf_hbm, df_hbm, *,
          sidx, didx, sidx_t, didx_t, rows_a, rows_b, rows_ta, rows_tb,
          isem, ga, gb, wa, wb):
        c = lax.axis_index("c")
        s = lax.axis_index("s")
        base = (c * NS + s) * GT

        def off_of(jj):
            return pl.multiple_of(base + jj * 128, 8)

        def idx_issue(jj, b4):
            off = off_of(jj)
            pltpu.async_copy(src_hbm.at[pl.ds(off, 128)], sidx[b4], isem[b4])
            pltpu.async_copy(dst_hbm.at[pl.ds(off, 128)], didx[b4], isem[b4])

        def idx_wait(b4):
            pltpu.make_async_copy(src_hbm.at[pl.ds(0, 128)], sidx[b4], isem[b4]).wait()
            pltpu.make_async_copy(dst_hbm.at[pl.ds(0, 128)], didx[b4], isem[b4]).wait()

        def gathers_issue(b4, b2):
            pltpu.async_copy(y_hbm.at[sidx[b4]], rows_a[b2], ga[b2])
            pltpu.async_copy(y_hbm.at[didx[b4]], rows_b[b2], gb[b2])

        def gathers_wait(b4, b2):
            pltpu.make_async_copy(y_hbm.at[sidx[b4]], rows_a[b2], ga[b2]).wait()
            pltpu.make_async_copy(y_hbm.at[didx[b4]], rows_b[b2], gb[b2]).wait()

        def writes_issue(jj, b2):
            off = off_of(jj)
            pltpu.async_copy(rows_a[b2], sf_hbm.at[pl.ds(off, 128)], wa[b2])
            pltpu.async_copy(rows_b[b2], df_hbm.at[pl.ds(off, 128)], wb[b2])

        def writes_wait(b2):
            pltpu.make_async_copy(rows_a[b2], sf_hbm.at[pl.ds(0, 128)], wa[b2]).wait()
            pltpu.make_async_copy(rows_b[b2], df_hbm.at[pl.ds(0, 128)], wb[b2]).wait()

        def B(jj, u, issue_idx=True, first=False, second=False):
            b2 = u % 2
            b4 = (2 + u) % 4 if not (first or second) else (0 if first else 1)
            if not (first or second):
                writes_wait(b2)
            if issue_idx:
                idx_issue(jj + 2, u % 4 if not (first or second)
                          else (2 if first else 3))
            idx_wait(b4)
            gathers_issue(b4, b2)
            if not first:
                b4p = (b4 + 3) % 4
                gathers_wait(b4p, 1 - b2)
                writes_issue(jj - 1, 1 - b2)

        idx_issue(0, 0)
        idx_issue(1, 1)
        B(0, 0, first=True)
        B(1, 1, second=True)

        @pl.loop(2, GCH - 4, step=4)
        def _(v):
            for u in range(4):
                B(v + u, u)

        B(GCH - 4, 0)
        B(GCH - 3, 1)
        B(GCH - 2, 2, issue_idx=False)
        B(GCH - 1, 3, issue_idx=False)

        gathers_wait((GCH - 1) % 4, 1)
        writes_issue(GCH - 1, 1)
        writes_wait(0)
        writes_wait(1)

        off = base + GCH * 128
        pltpu.sync_copy(src_hbm.at[pl.ds(off, GTAIL)], sidx_t)
        pltpu.sync_copy(dst_hbm.at[pl.ds(off, GTAIL)], didx_t)
        ca = pltpu.async_copy(y_hbm.at[sidx_t], rows_ta, ga[0])
        cb = pltpu.async_copy(y_hbm.at[didx_t], rows_tb, gb[0])
        ca.wait()
        pltpu.sync_copy(rows_ta, sf_hbm.at[pl.ds(off, GTAIL)])
        cb.wait()
        pltpu.sync_copy(rows_tb, df_hbm.at[pl.ds(off, GTAIL)])

    return k(y, src, dst)


def _tc_dense(x, part, W_s, W_n, b, relu):
    R = 1000
    part3 = part.reshape(2, NP, D)
    b2d = b.reshape(1, D)

    def body(x_ref, p0_ref, p1_ref, ws_ref, wn_ref, b_ref, o_ref):
        acc = jnp.dot(x_ref[...], ws_ref[...], preferred_element_type=jnp.float32)
        acc = acc + jnp.dot(p0_ref[0] + p1_ref[0], wn_ref[...],
                            preferred_element_type=jnp.float32)
        acc = acc + b_ref[...]
        if relu:
            acc = jnp.maximum(acc, 0.0)
        o_ref[...] = acc

    return pl.pallas_call(
        body,
        grid=(N // R,),
        in_specs=[
            pl.BlockSpec((R, D), lambda i: (i, 0)),
            pl.BlockSpec((1, R, D), lambda i: (0, i, 0)),
            pl.BlockSpec((1, R, D), lambda i: (1, i, 0)),
            pl.BlockSpec((D, D), lambda i: (0, 0)),
            pl.BlockSpec((D, D), lambda i: (0, 0)),
            pl.BlockSpec((1, D), lambda i: (0, 0)),
        ],
        out_specs=pl.BlockSpec((R, D), lambda i: (i, 0)),
        out_shape=jax.ShapeDtypeStruct((N, D), jnp.float32),
    )(x, part3, part3, W_s, W_n, b2d)


def kernel(x, edge_index, W_self1, W_neigh1, b1, W_self2, W_neigh2, b2):
    src = edge_index[0].astype(jnp.int32)
    dst = edge_index[1].astype(jnp.int32)
    z2 = jnp.zeros((NP, D), jnp.float32)
    z1 = jnp.zeros((NP,), jnp.float32)

    part1, inv = _sc_agg_layer1(x, src, dst, z2, z1)
    h = _tc_dense(x, part1, W_self1, W_neigh1, b1, relu=True)
    part2 = _sc_agg_layer2(h, src, dst, inv, z2)
    out2 = _tc_dense(h, part2, W_self2, W_neigh2, b2, relu=False)
    src_feat, dst_feat = _sc_gather_out(out2, src, dst)
    return (src_feat, dst_feat)


# speedup vs baseline: 9.2058x; 1.0287x over previous
import functools

import jax
import jax.numpy as jnp
from jax import lax
from jax.experimental import pallas as pl
from jax.experimental.pallas import tpu as pltpu
from jax.experimental.pallas import tpu_sc as plsc

N = 10000
E = 320000
D = 128
NC = 2
NS = 16
NP = 10240
RPT = NP // NS

EC = E // NC
ET = EC // NS
FCH, FTAIL = ET // 128, ET % 128
GT = E // (NC * NS)
GCH = 2 * (GT // 128)
GTAIL = GT % 128

_mesh = plsc.VectorSubcoreMesh(core_axis_name="c", subcore_axis_name="s")

_IDX = lambda n: [pltpu.VMEM((128,), jnp.int32) for _ in range(n)]
_SEM = lambda n: [pltpu.SemaphoreType.DMA for _ in range(n)]
_ROWS = lambda n: [pltpu.VMEM((128, D), jnp.float32) for _ in range(n)]


def _fill_ones(ref, n):
    @pl.loop(0, n // 16)
    def _(i):
        ref[pl.ds(i * 16, 16)] = jnp.ones((16,), jnp.float32)


def _agg_pipeline(feat_hbm, src_hbm, dst_hbm, agg_sh, deg_sh, sidx, didx,
                  rows, ones_v, isem, gsem, ssem, dgsem, sidx_t, didx_t,
                  rows_t, ones_t, f_base, with_deg):
    def idx_issue(jj, b4):
        off = pl.multiple_of(f_base + jj * 128, 8)
        pltpu.async_copy(src_hbm.at[pl.ds(off, 128)], sidx[b4], isem[b4])
        pltpu.async_copy(dst_hbm.at[pl.ds(off, 128)], didx[b4], isem[b4])

    def idx_wait(b4):
        pltpu.make_async_copy(src_hbm.at[pl.ds(0, 128)], sidx[b4], isem[b4]).wait()
        pltpu.make_async_copy(dst_hbm.at[pl.ds(0, 128)], didx[b4], isem[b4]).wait()

    def scat_issue(b4, b2):
        pltpu.async_copy(rows[b2], agg_sh.at[didx[b4]], ssem[b2], add=True)
        if with_deg:
            pltpu.async_copy(ones_v, deg_sh.at[didx[b4]], dgsem[b2], add=True)

    def scat_wait(b4, b2):
        pltpu.make_async_copy(rows[b2], agg_sh.at[didx[b4]], ssem[b2]).wait()
        if with_deg:
            pltpu.make_async_copy(ones_v, deg_sh.at[didx[b4]], dgsem[b2]).wait()

    def B(jj, u, issue_idx=True, first=False, second=False):
        b2 = u % 2
        b4 = (2 + u) % 4 if not (first or second) else (0 if first else 1)
        if not (first or second):
            scat_wait(u % 4, b2)
        if issue_idx:
            idx_issue(jj + 2, u % 4 if not (first or second)
                      else (2 if first else 3))
        idx_wait(b4)
        pltpu.async_copy(feat_hbm.at[sidx[b4]], rows[b2], gsem[b2])
        if not first:
            b4p = (b4 + 3) % 4
            pltpu.make_async_copy(feat_hbm.at[sidx[b4p]], rows[1 - b2],
                                  gsem[1 - b2]).wait()
            scat_issue(b4p, 1 - b2)

    idx_issue(0, 0)
    idx_issue(1, 1)
    B(0, 0, first=True)
    B(1, 1, second=True)

    @pl.loop(2, FCH - 4, step=4)
    def _(v):
        for u in range(4):
            B(v + u, u)

    B(FCH - 4, 0)
    B(FCH - 3, 1)
    B(FCH - 2, 2, issue_idx=False)
    B(FCH - 1, 3, issue_idx=False)

    pltpu.make_async_copy(feat_hbm.at[sidx[(FCH - 1) % 4]], rows[1],
                          gsem[1]).wait()
    scat_issue((FCH - 1) % 4, 1)
    scat_wait((FCH - 2) % 4, 0)
    scat_wait((FCH - 1) % 4, 1)

    off = f_base + FCH * 128
    pltpu.sync_copy(src_hbm.at[pl.ds(off, FTAIL)], sidx_t)
    pltpu.sync_copy(dst_hbm.at[pl.ds(off, FTAIL)], didx_t)
    pltpu.async_copy(feat_hbm.at[sidx_t], rows_t, gsem[0]).wait()
    pltpu.sync_copy(rows_t, agg_sh.at[didx_t], add=True)
    if with_deg:
        pltpu.sync_copy(ones_t, deg_sh.at[didx_t], add=True)


def _sc_agg(feat, src, dst, z2, z1, with_deg):
    out_type = [jax.ShapeDtypeStruct((2 * NP, D), jnp.float32)]
    if with_deg:
        out_type.append(jax.ShapeDtypeStruct((2 * NP,), jnp.float32))

    @functools.partial(
        pl.kernel,
        out_type=tuple(out_type),
        mesh=_mesh,
        scratch_types=dict(
            agg_sh=pltpu.VMEM_SHARED((NP, D), jnp.float32),
            deg_sh=pltpu.VMEM_SHARED((NP,), jnp.float32),
            sidx=_IDX(4), didx=_IDX(4),
            sidx_t=pltpu.VMEM((FTAIL,), jnp.int32),
            didx_t=pltpu.VMEM((FTAIL,), jnp.int32),
            ones_v=pltpu.VMEM((128,), jnp.float32),
            ones_t=pltpu.VMEM((FTAIL,), jnp.float32),
            rows=_ROWS(2),
            rows_t=pltpu.VMEM((FTAIL, D), jnp.float32),
            isem=_SEM(4), gsem=_SEM(2), ssem=_SEM(2), dgsem=_SEM(2),
        ),
    )
    def k(feat_hbm, src_hbm, dst_hbm, z2_hbm, z1_hbm, *out_and_scratch,
          agg_sh, deg_sh, sidx, didx, sidx_t, didx_t, ones_v, ones_t,
          rows, rows_t, isem, gsem, ssem, dgsem):
        if with_deg:
            part_hbm, degp_hbm = out_and_scratch
        else:
            (part_hbm,) = out_and_scratch
            degp_hbm = None
        c = lax.axis_index("c")
        s = lax.axis_index("s")
        r0 = s * RPT

        pltpu.sync_copy(z2_hbm.at[pl.ds(r0, RPT)], agg_sh.at[pl.ds(r0, RPT)])
        if with_deg:
            pltpu.sync_copy(z1_hbm.at[pl.ds(r0, RPT)], deg_sh.at[pl.ds(r0, RPT)])
            _fill_ones(ones_v, 128)
            _fill_ones(ones_t, FTAIL)
        plsc.subcore_barrier()

        _agg_pipeline(feat_hbm, src_hbm, dst_hbm, agg_sh, deg_sh, sidx, didx,
                      rows, ones_v, isem, gsem, ssem, dgsem, sidx_t, didx_t,
                      rows_t, ones_t, c * EC + s * ET, with_deg)
        plsc.subcore_barrier()

        pltpu.sync_copy(agg_sh.at[pl.ds(r0, RPT)],
                        part_hbm.at[pl.ds(c * NP + r0, RPT)])
        if with_deg:
            pltpu.sync_copy(deg_sh.at[pl.ds(r0, RPT)],
                            degp_hbm.at[pl.ds(c * NP + r0, RPT)])

    return k(feat, src, dst, z2, z1)


def _sc_gather_out(y, src, dst):
    @functools.partial(
        pl.kernel,
        out_type=(
            jax.ShapeDtypeStruct((E, D), jnp.float32),
            jax.ShapeDtypeStruct((E, D), jnp.float32),
        ),
        mesh=_mesh,
        scratch_types=dict(
            idx=_IDX(4),
            sidx_t=pltpu.VMEM((GTAIL,), jnp.int32),
            didx_t=pltpu.VMEM((GTAIL,), jnp.int32),
            rows=_ROWS(4),
            rows_ta=pltpu.VMEM((GTAIL, D), jnp.float32),
            rows_tb=pltpu.VMEM((GTAIL, D), jnp.float32),
            isem=_SEM(4), gsem=_SEM(4), wsem=_SEM(4),
        ),
    )
    def k(y_hbm, src_hbm, dst_hbm, sf_hbm, df_hbm, *,
          idx, sidx_t, didx_t, rows, rows_ta, rows_tb, isem, gsem, wsem):
        c = lax.axis_index("c")
        s = lax.axis_index("s")
        base = (c * NS + s) * GT

        def off_of(jj):
            return pl.multiple_of(base + (jj // 2) * 128, 8)

        def idx_issue(jj, b4, even):
            ref = src_hbm if even else dst_hbm
            pltpu.async_copy(ref.at[pl.ds(off_of(jj), 128)], idx[b4], isem[b4])

        def idx_wait(b4):
            pltpu.make_async_copy(src_hbm.at[pl.ds(0, 128)], idx[b4],
                                  isem[b4]).wait()

        def write_issue(jj, b4, even):
            out = sf_hbm if even else df_hbm
            pltpu.async_copy(rows[b4], out.at[pl.ds(off_of(jj), 128)], wsem[b4])

        def write_wait(b4, even):
            out = sf_hbm if even else df_hbm
            pltpu.make_async_copy(rows[b4], out.at[pl.ds(0, 128)],
                                  wsem[b4]).wait()

        def B(jj, b4, even):
            if not isinstance(jj, int) or jj >= 4:
                write_wait(b4, even)
            idx_wait(b4)
            pltpu.async_copy(y_hbm.at[idx[b4]], rows[b4], gsem[b4])
            if not isinstance(jj, int) or jj >= 2:
                jp4 = (b4 + 2) % 4
                pltpu.make_async_copy(y_hbm.at[idx[jp4]], rows[jp4],
                                      gsem[jp4]).wait()
                write_issue(jj - 2, jp4, even)
            if not isinstance(jj, int):
                idx_issue(jj + 2, (b4 + 2) % 4, even)
            elif jj + 2 < GCH:
                idx_issue(jj + 2, (b4 + 2) % 4, even)

        idx_issue(0, 0, True)
        idx_issue(1, 1, False)
        for jj in range(4):
            B(jj, jj % 4, jj % 2 == 0)

        @pl.loop(4, 152, step=4)
        def _(v):
            for u in range(4):
                B(v + u, u, u % 2 == 0)

        for jj in range(152, GCH):
            B(jj, jj % 4, jj % 2 == 0)

        for jj in (GCH - 2, GCH - 1):
            b4, even = jj % 4, jj % 2 == 0
            pltpu.make_async_copy(y_hbm.at[idx[b4]], rows[b4], gsem[b4]).wait()
            write_issue(jj, b4, even)
        for jj in range(GCH - 4, GCH):
            write_wait(jj % 4, jj % 2 == 0)

        off = base + (GCH // 2) * 128
        pltpu.sync_copy(src_hbm.at[pl.ds(off, GTAIL)], sidx_t)
        pltpu.sync_copy(dst_hbm.at[pl.ds(off, GTAIL)], didx_t)
        ca = pltpu.async_copy(y_hbm.at[sidx_t], rows_ta, gsem[0])
        cb = pltpu.async_copy(y_hbm.at[didx_t], rows_tb, gsem[1])
        ca.wait()
        pltpu.sync_copy(rows_ta, sf_hbm.at[pl.ds(off, GTAIL)])
        cb.wait()
        pltpu.sync_copy(rows_tb, df_hbm.at[pl.ds(off, GTAIL)])

    return k(y, src, dst)


def _tc_dense(x, part, degp, W_s, W_n, b, relu):
    R = 1000
    part3 = part.reshape(2, NP, D)
    deg3 = degp.reshape(2, NP, 1)
    b2d = b.reshape(1, D)

    def body(x_ref, p0_ref, p1_ref, d_ref, ws_ref, wn_ref, b_ref, o_ref):
        dsum = d_ref[0] + d_ref[1]
        scale = 1.0 / jnp.maximum(dsum, 1.0)
        agg = (p0_ref[0] + p1_ref[0]) * scale
        acc = jnp.dot(x_ref[...], ws_ref[...], preferred_element_type=jnp.float32)
        acc = acc + jnp.dot(agg, wn_ref[...], preferred_element_type=jnp.float32)
        acc = acc + b_ref[...]
        if relu:
            acc = jnp.maximum(acc, 0.0)
        o_ref[...] = acc

    return pl.pallas_call(
        body,
        grid=(N // R,),
        in_specs=[
            pl.BlockSpec((R, D), lambda i: (i, 0)),
            pl.BlockSpec((1, R, D), lambda i: (0, i, 0)),
            pl.BlockSpec((1, R, D), lambda i: (1, i, 0)),
            pl.BlockSpec((2, R, 1), lambda i: (0, i, 0)),
            pl.BlockSpec((D, D), lambda i: (0, 0)),
            pl.BlockSpec((D, D), lambda i: (0, 0)),
            pl.BlockSpec((1, D), lambda i: (0, 0)),
        ],
        out_specs=pl.BlockSpec((R, D), lambda i: (i, 0)),
        out_shape=jax.ShapeDtypeStruct((N, D), jnp.float32),
    )(x, part3, part3, deg3, W_s, W_n, b2d)


def kernel(x, edge_index, W_self1, W_neigh1, b1, W_self2, W_neigh2, b2):
    src = edge_index[0].astype(jnp.int32)
    dst = edge_index[1].astype(jnp.int32)
    z2 = jnp.zeros((NP, D), jnp.float32)
    z1 = jnp.zeros((NP,), jnp.float32)

    part1, degp = _sc_agg(x, src, dst, z2, z1, with_deg=True)
    h = _tc_dense(x, part1, degp, W_self1, W_neigh1, b1, relu=True)
    (part2,) = _sc_agg(h, src, dst, z2, z1, with_deg=False)
    out2 = _tc_dense(h, part2, degp, W_self2, W_neigh2, b2, relu=False)
    src_feat, dst_feat = _sc_gather_out(out2, src, dst)
    return (src_feat, dst_feat)


# speedup vs baseline: 9.4450x; 1.0260x over previous
import functools

import jax
import jax.numpy as jnp
from jax import lax
from jax.experimental import pallas as pl
from jax.experimental.pallas import tpu as pltpu
from jax.experimental.pallas import tpu_sc as plsc

N = 10000
E = 320000
D = 128
NC = 2
NS = 16
NP = 10240
RPT = NP // NS

EC = E // NC
ET = EC // NS
CS = 64
FCH, FTAIL = ET // CS, ET % CS
GT = E // (NC * NS)
GCH = 2 * (GT // 128)
GTAIL = GT % 128

_mesh = plsc.VectorSubcoreMesh(core_axis_name="c", subcore_axis_name="s")

_IDX = lambda n: [pltpu.VMEM((128,), jnp.int32) for _ in range(n)]
_SEM = lambda n: [pltpu.SemaphoreType.DMA for _ in range(n)]
_ROWS = lambda n: [pltpu.VMEM((128, D), jnp.float32) for _ in range(n)]


def _fill_ones(ref, n):
    @pl.loop(0, n // 16)
    def _(i):
        ref[pl.ds(i * 16, 16)] = jnp.ones((16,), jnp.float32)


def _agg_pipeline(feat_hbm, src_hbm, dst_hbm, agg_sh, deg_sh, sidx, didx,
                  rows, ones_v, isem, gsem, ssem, dgsem, sidx_t, didx_t,
                  rows_t, ones_t, f_base, with_deg):
    def idx_issue(jj, b4, b8):
        off = pl.multiple_of(f_base + jj * CS, 8)
        pltpu.async_copy(src_hbm.at[pl.ds(off, CS)], sidx[b4], isem[b4])
        pltpu.async_copy(dst_hbm.at[pl.ds(off, CS)], didx[b8], isem[b4])

    def idx_wait(b4, b8):
        pltpu.make_async_copy(src_hbm.at[pl.ds(0, CS)], sidx[b4], isem[b4]).wait()
        pltpu.make_async_copy(dst_hbm.at[pl.ds(0, CS)], didx[b8], isem[b4]).wait()

    def scat_issue(b8, b4):
        pltpu.async_copy(rows[b4], agg_sh.at[didx[b8]], ssem[b4], add=True)
        if with_deg:
            pltpu.async_copy(ones_v, deg_sh.at[didx[b8]], dgsem[b4], add=True)

    def scat_wait(b8, b4):
        pltpu.make_async_copy(rows[b4], agg_sh.at[didx[b8]], ssem[b4]).wait()
        if with_deg:
            pltpu.make_async_copy(ones_v, deg_sh.at[didx[b8]], dgsem[b4]).wait()

    def gather_wait(b4):
        pltpu.make_async_copy(feat_hbm.at[sidx[b4]], rows[b4], gsem[b4]).wait()

    def B(jj, b4, b8):
        traced = not isinstance(jj, int)
        if traced or jj >= 3:
            scat_wait((b8 + 5) % 8, (b4 + 1) % 4)
        idx_wait(b4, b8)
        pltpu.async_copy(feat_hbm.at[sidx[b4]], rows[b4], gsem[b4])
        if traced or jj >= 2:
            jp4, jp8 = (b4 + 2) % 4, (b8 + 6) % 8
            gather_wait(jp4)
            scat_issue(jp8, jp4)
        if traced or jj + 2 < FCH:
            idx_issue(jj + 2, (b4 + 2) % 4, (b8 + 2) % 8)

    idx_issue(0, 0, 0)
    idx_issue(1, 1, 1)
    for jj in range(4):
        B(jj, jj % 4, jj % 8)

    @pl.loop(4, FCH - 8, step=8)
    def _(v):
        for u in range(8):
            B(v + u, (4 + u) % 4, (4 + u) % 8)

    for jj in range(FCH - 8, FCH):
        B(jj, jj % 4, jj % 8)

    scat_wait((FCH - 3) % 8, (FCH - 3) % 4)
    for jj in (FCH - 2, FCH - 1):
        gather_wait(jj % 4)
        scat_issue(jj % 8, jj % 4)
        scat_wait(jj % 8, jj % 4)

    off = f_base + FCH * CS
    pltpu.sync_copy(src_hbm.at[pl.ds(off, FTAIL)], sidx_t)
    pltpu.sync_copy(dst_hbm.at[pl.ds(off, FTAIL)], didx_t)
    pltpu.async_copy(feat_hbm.at[sidx_t], rows_t, gsem[0]).wait()
    pltpu.sync_copy(rows_t, agg_sh.at[didx_t], add=True)
    if with_deg:
        pltpu.sync_copy(ones_t, deg_sh.at[didx_t], add=True)


def _sc_agg(feat, src, dst, z2, z1, with_deg):
    out_type = [jax.ShapeDtypeStruct((2 * NP, D), jnp.float32)]
    if with_deg:
        out_type.append(jax.ShapeDtypeStruct((2 * NP,), jnp.float32))

    @functools.partial(
        pl.kernel,
        out_type=tuple(out_type),
        mesh=_mesh,
        scratch_types=dict(
            agg_sh=pltpu.VMEM_SHARED((NP, D), jnp.float32),
            deg_sh=pltpu.VMEM_SHARED((NP,), jnp.float32),
            sidx=[pltpu.VMEM((CS,), jnp.int32) for _ in range(4)],
            didx=[pltpu.VMEM((CS,), jnp.int32) for _ in range(8)],
            sidx_t=pltpu.VMEM((FTAIL,), jnp.int32),
            didx_t=pltpu.VMEM((FTAIL,), jnp.int32),
            ones_v=pltpu.VMEM((CS,), jnp.float32),
            ones_t=pltpu.VMEM((FTAIL,), jnp.float32),
            rows=[pltpu.VMEM((CS, D), jnp.float32) for _ in range(4)],
            rows_t=pltpu.VMEM((FTAIL, D), jnp.float32),
            isem=_SEM(4), gsem=_SEM(4), ssem=_SEM(4), dgsem=_SEM(4),
        ),
    )
    def k(feat_hbm, src_hbm, dst_hbm, z2_hbm, z1_hbm, *out_and_scratch,
          agg_sh, deg_sh, sidx, didx, sidx_t, didx_t, ones_v, ones_t,
          rows, rows_t, isem, gsem, ssem, dgsem):
        if with_deg:
            part_hbm, degp_hbm = out_and_scratch
        else:
            (part_hbm,) = out_and_scratch
            degp_hbm = None
        c = lax.axis_index("c")
        s = lax.axis_index("s")
        r0 = s * RPT

        pltpu.sync_copy(z2_hbm.at[pl.ds(r0, RPT)], agg_sh.at[pl.ds(r0, RPT)])
        if with_deg:
            pltpu.sync_copy(z1_hbm.at[pl.ds(r0, RPT)], deg_sh.at[pl.ds(r0, RPT)])
            _fill_ones(ones_v, CS)
            _fill_ones(ones_t, FTAIL)
        plsc.subcore_barrier()

        _agg_pipeline(feat_hbm, src_hbm, dst_hbm, agg_sh, deg_sh, sidx, didx,
                      rows, ones_v, isem, gsem, ssem, dgsem, sidx_t, didx_t,
                      rows_t, ones_t, c * EC + s * ET, with_deg)
        plsc.subcore_barrier()

        pltpu.sync_copy(agg_sh.at[pl.ds(r0, RPT)],
                        part_hbm.at[pl.ds(c * NP + r0, RPT)])
        if with_deg:
            pltpu.sync_copy(deg_sh.at[pl.ds(r0, RPT)],
                            degp_hbm.at[pl.ds(c * NP + r0, RPT)])

    return k(feat, src, dst, z2, z1)


def _sc_gather_out(y, src, dst):
    @functools.partial(
        pl.kernel,
        out_type=(
            jax.ShapeDtypeStruct((E, D), jnp.float32),
            jax.ShapeDtypeStruct((E, D), jnp.float32),
        ),
        mesh=_mesh,
        scratch_types=dict(
            idx=_IDX(4),
            sidx_t=pltpu.VMEM((GTAIL,), jnp.int32),
            didx_t=pltpu.VMEM((GTAIL,), jnp.int32),
            rows=_ROWS(4),
            rows_ta=pltpu.VMEM((GTAIL, D), jnp.float32),
            rows_tb=pltpu.VMEM((GTAIL, D), jnp.float32),
            isem=_SEM(4), gsem=_SEM(4), wsem=_SEM(4),
        ),
    )
    def k(y_hbm, src_hbm, dst_hbm, sf_hbm, df_hbm, *,
          idx, sidx_t, didx_t, rows, rows_ta, rows_tb, isem, gsem, wsem):
        c = lax.axis_index("c")
        s = lax.axis_index("s")
        base = (c * NS + s) * GT

        def off_of(jj):
            return pl.multiple_of(base + (jj // 2) * 128, 8)

        def idx_issue(jj, b4, even):
            ref = src_hbm if even else dst_hbm
            pltpu.async_copy(ref.at[pl.ds(off_of(jj), 128)], idx[b4], isem[b4])

        def idx_wait(b4):
            pltpu.make_async_copy(src_hbm.at[pl.ds(0, 128)], idx[b4],
                                  isem[b4]).wait()

        def write_issue(jj, b4, even):
            out = sf_hbm if even else df_hbm
            pltpu.async_copy(rows[b4], out.at[pl.ds(off_of(jj), 128)], wsem[b4])

        def write_wait(b4, even):
            out = sf_hbm if even else df_hbm
            pltpu.make_async_copy(rows[b4], out.at[pl.ds(0, 128)],
                                  wsem[b4]).wait()

        def B(jj, b4, even):
            if not isinstance(jj, int) or jj >= 4:
                write_wait(b4, even)
            idx_wait(b4)
            pltpu.async_copy(y_hbm.at[idx[b4]], rows[b4], gsem[b4])
            if not isinstance(jj, int) or jj >= 2:
                jp4 = (b4 + 2) % 4
                pltpu.make_async_copy(y_hbm.at[idx[jp4]], rows[jp4],
                                      gsem[jp4]).wait()
                write_issue(jj - 2, jp4, even)
            if not isinstance(jj, int):
                idx_issue(jj + 2, (b4 + 2) % 4, even)
            elif jj + 2 < GCH:
                idx_issue(jj + 2, (b4 + 2) % 4, even)

        idx_issue(0, 0, True)
        idx_issue(1, 1, False)
        for jj in range(4):
            B(jj, jj % 4, jj % 2 == 0)

        @pl.loop(4, 152, step=4)
        def _(v):
            for u in range(4):
                B(v + u, u, u % 2 == 0)

        for jj in range(152, GCH):
            B(jj, jj % 4, jj % 2 == 0)

        for jj in (GCH - 2, GCH - 1):
            b4, even = jj % 4, jj % 2 == 0
            pltpu.make_async_copy(y_hbm.at[idx[b4]], rows[b4], gsem[b4]).wait()
            write_issue(jj, b4, even)
        for jj in range(GCH - 4, GCH):
            write_wait(jj % 4, jj % 2 == 0)

        off = base + (GCH // 2) * 128
        pltpu.sync_copy(src_hbm.at[pl.ds(off, GTAIL)], sidx_t)
        pltpu.sync_copy(dst_hbm.at[pl.ds(off, GTAIL)], didx_t)
        ca = pltpu.async_copy(y_hbm.at[sidx_t], rows_ta, gsem[0])
        cb = pltpu.async_copy(y_hbm.at[didx_t], rows_tb, gsem[1])
        ca.wait()
        pltpu.sync_copy(rows_ta, sf_hbm.at[pl.ds(off, GTAIL)])
        cb.wait()
        pltpu.sync_copy(rows_tb, df_hbm.at[pl.ds(off, GTAIL)])

    return k(y, src, dst)


def _tc_dense(x, part, degp, W_s, W_n, b, relu):
    R = 1000
    part3 = part.reshape(2, NP, D)
    deg3 = degp.reshape(2, NP, 1)
    b2d = b.reshape(1, D)

    def body(x_ref, p0_ref, p1_ref, d_ref, ws_ref, wn_ref, b_ref, o_ref):
        dsum = d_ref[0] + d_ref[1]
        scale = 1.0 / jnp.maximum(dsum, 1.0)
        agg = (p0_ref[0] + p1_ref[0]) * scale
        acc = jnp.dot(x_ref[...], ws_ref[...], preferred_element_type=jnp.float32)
        acc = acc + jnp.dot(agg, wn_ref[...], preferred_element_type=jnp.float32)
        acc = acc + b_ref[...]
        if relu:
            acc = jnp.maximum(acc, 0.0)
        o_ref[...] = acc

    return pl.pallas_call(
        body,
        grid=(N // R,),
        in_specs=[
            pl.BlockSpec((R, D), lambda i: (i, 0)),
            pl.BlockSpec((1, R, D), lambda i: (0, i, 0)),
            pl.BlockSpec((1, R, D), lambda i: (1, i, 0)),
            pl.BlockSpec((2, R, 1), lambda i: (0, i, 0)),
            pl.BlockSpec((D, D), lambda i: (0, 0)),
            pl.BlockSpec((D, D), lambda i: (0, 0)),
            pl.BlockSpec((1, D), lambda i: (0, 0)),
        ],
        out_specs=pl.BlockSpec((R, D), lambda i: (i, 0)),
        out_shape=jax.ShapeDtypeStruct((N, D), jnp.float32),
    )(x, part3, part3, deg3, W_s, W_n, b2d)


def kernel(x, edge_index, W_self1, W_neigh1, b1, W_self2, W_neigh2, b2):
    src = edge_index[0].astype(jnp.int32)
    dst = edge_index[1].astype(jnp.int32)
    z2 = jnp.zeros((NP, D), jnp.float32)
    z1 = jnp.zeros((NP,), jnp.float32)

    part1, degp = _sc_agg(x, src, dst, z2, z1, with_deg=True)
    h = _tc_dense(x, part1, degp, W_self1, W_neigh1, b1, relu=True)
    (part2,) = _sc_agg(h, src, dst, z2, z1, with_deg=False)
    out2 = _tc_dense(h, part2, degp, W_self2, W_neigh2, b2, relu=False)
    src_feat, dst_feat = _sc_gather_out(out2, src, dst)
    return (src_feat, dst_feat)


# speedup vs baseline: 11.8220x; 1.2517x over previous
import functools

import jax
import jax.numpy as jnp
from jax import lax
from jax.experimental import pallas as pl
from jax.experimental.pallas import tpu as pltpu
from jax.experimental.pallas import tpu_sc as plsc

N = 10000
E = 320000
D = 128
NC = 2
NS = 16
NP = 10240
RPT = NP // NS

EC = E // NC
ET = EC // NS
CS = 64
FCH, FTAIL = ET // CS, ET % CS
GT = E // (NC * NS)
GCS = 64
GCH = 2 * (GT // GCS)
GTAIL = GT % GCS
YRT = 624

_mesh = plsc.VectorSubcoreMesh(core_axis_name="c", subcore_axis_name="s")

_IDX = lambda n: [pltpu.VMEM((128,), jnp.int32) for _ in range(n)]
_SEM = lambda n: [pltpu.SemaphoreType.DMA for _ in range(n)]
_ROWS = lambda n: [pltpu.VMEM((128, D), jnp.float32) for _ in range(n)]


def _fill_ones(ref, n):
    @pl.loop(0, n // 16)
    def _(i):
        ref[pl.ds(i * 16, 16)] = jnp.ones((16,), jnp.float32)


def _agg_pipeline(feat_hbm, src_hbm, dst_hbm, agg_sh, deg_sh, sidx, didx,
                  rows, ones_v, isem, gsem, ssem, dgsem, sidx_t, didx_t,
                  rows_t, ones_t, f_base, with_deg):
    def idx_issue(jj, b4, b8):
        off = pl.multiple_of(f_base + jj * CS, 8)
        pltpu.async_copy(src_hbm.at[pl.ds(off, CS)], sidx[b4], isem[b4])
        pltpu.async_copy(dst_hbm.at[pl.ds(off, CS)], didx[b8], isem[b4])

    def idx_wait(b4, b8):
        pltpu.make_async_copy(src_hbm.at[pl.ds(0, CS)], sidx[b4], isem[b4]).wait()
        pltpu.make_async_copy(dst_hbm.at[pl.ds(0, CS)], didx[b8], isem[b4]).wait()

    def scat_issue(b8, b4):
        pltpu.async_copy(rows[b4], agg_sh.at[didx[b8]], ssem[b4], add=True)
        if with_deg:
            pltpu.async_copy(ones_v, deg_sh.at[didx[b8]], dgsem[b4], add=True)

    def scat_wait(b8, b4):
        pltpu.make_async_copy(rows[b4], agg_sh.at[didx[b8]], ssem[b4]).wait()
        if with_deg:
            pltpu.make_async_copy(ones_v, deg_sh.at[didx[b8]], dgsem[b4]).wait()

    def gather_wait(b4):
        pltpu.make_async_copy(feat_hbm.at[sidx[b4]], rows[b4], gsem[b4]).wait()

    def B(jj, b4, b8):
        traced = not isinstance(jj, int)
        if traced or jj >= 3:
            scat_wait((b8 + 5) % 8, (b4 + 1) % 4)
        idx_wait(b4, b8)
        pltpu.async_copy(feat_hbm.at[sidx[b4]], rows[b4], gsem[b4])
        if traced or jj >= 2:
            jp4, jp8 = (b4 + 2) % 4, (b8 + 6) % 8
            gather_wait(jp4)
            scat_issue(jp8, jp4)
        if traced or jj + 2 < FCH:
            idx_issue(jj + 2, (b4 + 2) % 4, (b8 + 2) % 8)

    idx_issue(0, 0, 0)
    idx_issue(1, 1, 1)
    for jj in range(4):
        B(jj, jj % 4, jj % 8)

    @pl.loop(4, FCH - 8, step=8)
    def _(v):
        for u in range(8):
            B(v + u, (4 + u) % 4, (4 + u) % 8)

    for jj in range(FCH - 8, FCH):
        B(jj, jj % 4, jj % 8)

    scat_wait((FCH - 3) % 8, (FCH - 3) % 4)
    for jj in (FCH - 2, FCH - 1):
        gather_wait(jj % 4)
        scat_issue(jj % 8, jj % 4)
        scat_wait(jj % 8, jj % 4)

    off = f_base + FCH * CS
    pltpu.sync_copy(src_hbm.at[pl.ds(off, FTAIL)], sidx_t)
    pltpu.sync_copy(dst_hbm.at[pl.ds(off, FTAIL)], didx_t)
    pltpu.async_copy(feat_hbm.at[sidx_t], rows_t, gsem[0]).wait()
    pltpu.sync_copy(rows_t, agg_sh.at[didx_t], add=True)
    if with_deg:
        pltpu.sync_copy(ones_t, deg_sh.at[didx_t], add=True)


def _sc_agg(feat, src, dst, z2, z1, with_deg):
    out_type = [jax.ShapeDtypeStruct((2 * NP, D), jnp.float32)]
    if with_deg:
        out_type.append(jax.ShapeDtypeStruct((2 * NP,), jnp.float32))

    @functools.partial(
        pl.kernel,
        out_type=tuple(out_type),
        mesh=_mesh,
        scratch_types=dict(
            agg_sh=pltpu.VMEM_SHARED((NP, D), jnp.float32),
            deg_sh=pltpu.VMEM_SHARED((NP,), jnp.float32),
            sidx=[pltpu.VMEM((CS,), jnp.int32) for _ in range(4)],
            didx=[pltpu.VMEM((CS,), jnp.int32) for _ in range(8)],
            sidx_t=pltpu.VMEM((FTAIL,), jnp.int32),
            didx_t=pltpu.VMEM((FTAIL,), jnp.int32),
            ones_v=pltpu.VMEM((CS,), jnp.float32),
            ones_t=pltpu.VMEM((FTAIL,), jnp.float32),
            rows=[pltpu.VMEM((CS, D), jnp.float32) for _ in range(4)],
            rows_t=pltpu.VMEM((FTAIL, D), jnp.float32),
            isem=_SEM(4), gsem=_SEM(4), ssem=_SEM(4), dgsem=_SEM(4),
        ),
    )
    def k(feat_hbm, src_hbm, dst_hbm, z2_hbm, z1_hbm, *out_and_scratch,
          agg_sh, deg_sh, sidx, didx, sidx_t, didx_t, ones_v, ones_t,
          rows, rows_t, isem, gsem, ssem, dgsem):
        if with_deg:
            part_hbm, degp_hbm = out_and_scratch
        else:
            (part_hbm,) = out_and_scratch
            degp_hbm = None
        c = lax.axis_index("c")
        s = lax.axis_index("s")
        r0 = s * RPT

        pltpu.sync_copy(z2_hbm.at[pl.ds(r0, RPT)], agg_sh.at[pl.ds(r0, RPT)])
        if with_deg:
            pltpu.sync_copy(z1_hbm.at[pl.ds(r0, RPT)], deg_sh.at[pl.ds(r0, RPT)])
            _fill_ones(ones_v, CS)
            _fill_ones(ones_t, FTAIL)
        plsc.subcore_barrier()

        _agg_pipeline(feat_hbm, src_hbm, dst_hbm, agg_sh, deg_sh, sidx, didx,
                      rows, ones_v, isem, gsem, ssem, dgsem, sidx_t, didx_t,
                      rows_t, ones_t, c * EC + s * ET, with_deg)
        plsc.subcore_barrier()

        pltpu.sync_copy(agg_sh.at[pl.ds(r0, RPT)],
                        part_hbm.at[pl.ds(c * NP + r0, RPT)])
        if with_deg:
            pltpu.sync_copy(deg_sh.at[pl.ds(r0, RPT)],
                            degp_hbm.at[pl.ds(c * NP + r0, RPT)])

    return k(feat, src, dst, z2, z1)


def _sc_gather_out(y, src, dst):
    @functools.partial(
        pl.kernel,
        out_type=(
            jax.ShapeDtypeStruct((E, D), jnp.float32),
            jax.ShapeDtypeStruct((E, D), jnp.float32),
        ),
        mesh=_mesh,
        scratch_types=dict(
            y_sh=pltpu.VMEM_SHARED((N, D), jnp.float32),
            idx=[pltpu.VMEM((GCS,), jnp.int32) for _ in range(4)],
            sidx_t=pltpu.VMEM((GTAIL,), jnp.int32),
            didx_t=pltpu.VMEM((GTAIL,), jnp.int32),
            rows=[pltpu.VMEM((GCS, D), jnp.float32) for _ in range(4)],
            rows_ta=pltpu.VMEM((GTAIL, D), jnp.float32),
            rows_tb=pltpu.VMEM((GTAIL, D), jnp.float32),
            isem=_SEM(4), gsem=_SEM(4), wsem=_SEM(4),
        ),
    )
    def k(y_hbm, src_hbm, dst_hbm, sf_hbm, df_hbm, *,
          y_sh, idx, sidx_t, didx_t, rows, rows_ta, rows_tb,
          isem, gsem, wsem):
        c = lax.axis_index("c")
        s = lax.axis_index("s")
        base = (c * NS + s) * GT

        yr = s * YRT
        pltpu.sync_copy(y_hbm.at[pl.ds(yr, YRT)], y_sh.at[pl.ds(yr, YRT)])

        @pl.when(s == NS - 1)
        def _():
            pltpu.sync_copy(y_hbm.at[pl.ds(NS * YRT, N - NS * YRT)],
                            y_sh.at[pl.ds(NS * YRT, N - NS * YRT)])

        plsc.subcore_barrier()

        def off_of(jj):
            return pl.multiple_of(base + (jj // 2) * GCS, 8)

        def idx_issue(jj, b4, even):
            ref = src_hbm if even else dst_hbm
            pltpu.async_copy(ref.at[pl.ds(off_of(jj), GCS)], idx[b4], isem[b4])

        def idx_wait(b4):
            pltpu.make_async_copy(src_hbm.at[pl.ds(0, GCS)], idx[b4],
                                  isem[b4]).wait()

        def write_issue(jj, b4, even):
            out = sf_hbm if even else df_hbm
            pltpu.async_copy(rows[b4], out.at[pl.ds(off_of(jj), GCS)], wsem[b4])

        def write_wait(b4, even):
            out = sf_hbm if even else df_hbm
            pltpu.make_async_copy(rows[b4], out.at[pl.ds(0, GCS)],
                                  wsem[b4]).wait()

        def B(jj, b4, even):
            if not isinstance(jj, int) or jj >= 4:
                write_wait(b4, even)
            idx_wait(b4)
            pltpu.async_copy(y_sh.at[idx[b4]], rows[b4], gsem[b4])
            if not isinstance(jj, int) or jj >= 2:
                jp4 = (b4 + 2) % 4
                pltpu.make_async_copy(y_sh.at[idx[jp4]], rows[jp4],
                                      gsem[jp4]).wait()
                write_issue(jj - 2, jp4, even)
            if not isinstance(jj, int):
                idx_issue(jj + 2, (b4 + 2) % 4, even)
            elif jj + 2 < GCH:
                idx_issue(jj + 2, (b4 + 2) % 4, even)

        idx_issue(0, 0, True)
        idx_issue(1, 1, False)
        for jj in range(4):
            B(jj, jj % 4, jj % 2 == 0)

        @pl.loop(4, GCH - 4, step=4)
        def _(v):
            for u in range(4):
                B(v + u, u, u % 2 == 0)

        for jj in range(GCH - 4, GCH):
            B(jj, jj % 4, jj % 2 == 0)

        for jj in (GCH - 2, GCH - 1):
            b4, even = jj % 4, jj % 2 == 0
            pltpu.make_async_copy(y_sh.at[idx[b4]], rows[b4], gsem[b4]).wait()
            write_issue(jj, b4, even)
        for jj in range(GCH - 4, GCH):
            write_wait(jj % 4, jj % 2 == 0)

        off = base + (GCH // 2) * GCS
        pltpu.sync_copy(src_hbm.at[pl.ds(off, GTAIL)], sidx_t)
        pltpu.sync_copy(dst_hbm.at[pl.ds(off, GTAIL)], didx_t)
        ca = pltpu.async_copy(y_sh.at[sidx_t], rows_ta, gsem[0])
        cb = pltpu.async_copy(y_sh.at[didx_t], rows_tb, gsem[1])
        ca.wait()
        pltpu.sync_copy(rows_ta, sf_hbm.at[pl.ds(off, GTAIL)])
        cb.wait()
        pltpu.sync_copy(rows_tb, df_hbm.at[pl.ds(off, GTAIL)])

    return k(y, src, dst)


def _tc_dense(x, part, degp, W_s, W_n, b, relu):
    R = 1000
    part3 = part.reshape(2, NP, D)
    deg3 = degp.reshape(2, NP, 1)
    b2d = b.reshape(1, D)

    def body(x_ref, p0_ref, p1_ref, d_ref, ws_ref, wn_ref, b_ref, o_ref):
        dsum = d_ref[0] + d_ref[1]
        scale = 1.0 / jnp.maximum(dsum, 1.0)
        agg = (p0_ref[0] + p1_ref[0]) * scale
        acc = jnp.dot(x_ref[...], ws_ref[...], preferred_element_type=jnp.float32)
        acc = acc + jnp.dot(agg, wn_ref[...], preferred_element_type=jnp.float32)
        acc = acc + b_ref[...]
        if relu:
            acc = jnp.maximum(acc, 0.0)
        o_ref[...] = acc

    return pl.pallas_call(
        body,
        grid=(N // R,),
        in_specs=[
            pl.BlockSpec((R, D), lambda i: (i, 0)),
            pl.BlockSpec((1, R, D), lambda i: (0, i, 0)),
            pl.BlockSpec((1, R, D), lambda i: (1, i, 0)),
            pl.BlockSpec((2, R, 1), lambda i: (0, i, 0)),
            pl.BlockSpec((D, D), lambda i: (0, 0)),
            pl.BlockSpec((D, D), lambda i: (0, 0)),
            pl.BlockSpec((1, D), lambda i: (0, 0)),
        ],
        out_specs=pl.BlockSpec((R, D), lambda i: (i, 0)),
        out_shape=jax.ShapeDtypeStruct((N, D), jnp.float32),
    )(x, part3, part3, deg3, W_s, W_n, b2d)


def kernel(x, edge_index, W_self1, W_neigh1, b1, W_self2, W_neigh2, b2):
    src = edge_index[0].astype(jnp.int32)
    dst = edge_index[1].astype(jnp.int32)
    z2 = jnp.zeros((NP, D), jnp.float32)
    z1 = jnp.zeros((NP,), jnp.float32)

    part1, degp = _sc_agg(x, src, dst, z2, z1, with_deg=True)
    h = _tc_dense(x, part1, degp, W_self1, W_neigh1, b1, relu=True)
    (part2,) = _sc_agg(h, src, dst, z2, z1, with_deg=False)
    out2 = _tc_dense(h, part2, degp, W_self2, W_neigh2, b2, relu=False)
    src_feat, dst_feat = _sc_gather_out(out2, src, dst)
    return (src_feat, dst_feat)


# speedup vs baseline: 11.8576x; 1.0030x over previous
import functools

import jax
import jax.numpy as jnp
from jax import lax
from jax.experimental import pallas as pl
from jax.experimental.pallas import tpu as pltpu
from jax.experimental.pallas import tpu_sc as plsc

N = 10000
E = 320000
D = 128
NC = 2
NS = 16
NP = 10240
RPT = NP // NS

EC = E // NC
ET = EC // NS
CS = 64
FCH, FTAIL = ET // CS, ET % CS
GT = E // (NC * NS)
GCS = 80
GCH = 2 * (GT // GCS)
GTAIL = GT % GCS
YRT = 624

_mesh = plsc.VectorSubcoreMesh(core_axis_name="c", subcore_axis_name="s")

_IDX = lambda n: [pltpu.VMEM((128,), jnp.int32) for _ in range(n)]
_SEM = lambda n: [pltpu.SemaphoreType.DMA for _ in range(n)]
_ROWS = lambda n: [pltpu.VMEM((128, D), jnp.float32) for _ in range(n)]


def _fill_ones(ref, n):
    @pl.loop(0, n // 16)
    def _(i):
        ref[pl.ds(i * 16, 16)] = jnp.ones((16,), jnp.float32)


def _agg_pipeline(feat_hbm, src_hbm, dst_hbm, agg_sh, deg_sh, sidx, didx,
                  rows, ones_v, isem, gsem, ssem, dgsem, sidx_t, didx_t,
                  rows_t, ones_t, f_base, with_deg):
    def idx_issue(jj, b4, b8):
        off = pl.multiple_of(f_base + jj * CS, 8)
        pltpu.async_copy(src_hbm.at[pl.ds(off, CS)], sidx[b4], isem[b4])
        pltpu.async_copy(dst_hbm.at[pl.ds(off, CS)], didx[b8], isem[b4])

    def idx_wait(b4, b8):
        pltpu.make_async_copy(src_hbm.at[pl.ds(0, CS)], sidx[b4], isem[b4]).wait()
        pltpu.make_async_copy(dst_hbm.at[pl.ds(0, CS)], didx[b8], isem[b4]).wait()

    def scat_issue(b8, b4):
        pltpu.async_copy(rows[b4], agg_sh.at[didx[b8]], ssem[b4], add=True)
        if with_deg:
            pltpu.async_copy(ones_v, deg_sh.at[didx[b8]], dgsem[b4], add=True)

    def scat_wait(b8, b4):
        pltpu.make_async_copy(rows[b4], agg_sh.at[didx[b8]], ssem[b4]).wait()
        if with_deg:
            pltpu.make_async_copy(ones_v, deg_sh.at[didx[b8]], dgsem[b4]).wait()

    def gather_wait(b4):
        pltpu.make_async_copy(feat_hbm.at[sidx[b4]], rows[b4], gsem[b4]).wait()

    def B(jj, b4, b8):
        traced = not isinstance(jj, int)
        if traced or jj >= 3:
            scat_wait((b8 + 5) % 8, (b4 + 1) % 4)
        idx_wait(b4, b8)
        pltpu.async_copy(feat_hbm.at[sidx[b4]], rows[b4], gsem[b4])
        if traced or jj >= 2:
            jp4, jp8 = (b4 + 2) % 4, (b8 + 6) % 8
            gather_wait(jp4)
            scat_issue(jp8, jp4)
        if traced or jj + 2 < FCH:
            idx_issue(jj + 2, (b4 + 2) % 4, (b8 + 2) % 8)

    idx_issue(0, 0, 0)
    idx_issue(1, 1, 1)
    for jj in range(4):
        B(jj, jj % 4, jj % 8)

    @pl.loop(4, FCH - 8, step=8)
    def _(v):
        for u in range(8):
            B(v + u, (4 + u) % 4, (4 + u) % 8)

    for jj in range(FCH - 8, FCH):
        B(jj, jj % 4, jj % 8)

    scat_wait((FCH - 3) % 8, (FCH - 3) % 4)
    for jj in (FCH - 2, FCH - 1):
        gather_wait(jj % 4)
        scat_issue(jj % 8, jj % 4)
        scat_wait(jj % 8, jj % 4)

    off = f_base + FCH * CS
    pltpu.sync_copy(src_hbm.at[pl.ds(off, FTAIL)], sidx_t)
    pltpu.sync_copy(dst_hbm.at[pl.ds(off, FTAIL)], didx_t)
    pltpu.async_copy(feat_hbm.at[sidx_t], rows_t, gsem[0]).wait()
    pltpu.sync_copy(rows_t, agg_sh.at[didx_t], add=True)
    if with_deg:
        pltpu.sync_copy(ones_t, deg_sh.at[didx_t], add=True)


def _sc_agg(feat, src, dst, z2, z1, with_deg):
    out_type = [jax.ShapeDtypeStruct((2 * NP, D), jnp.float32)]
    if with_deg:
        out_type.append(jax.ShapeDtypeStruct((2 * NP,), jnp.float32))

    @functools.partial(
        pl.kernel,
        out_type=tuple(out_type),
        mesh=_mesh,
        scratch_types=dict(
            agg_sh=pltpu.VMEM_SHARED((NP, D), jnp.float32),
            deg_sh=pltpu.VMEM_SHARED((NP,), jnp.float32),
            sidx=[pltpu.VMEM((CS,), jnp.int32) for _ in range(4)],
            didx=[pltpu.VMEM((CS,), jnp.int32) for _ in range(8)],
            sidx_t=pltpu.VMEM((FTAIL,), jnp.int32),
            didx_t=pltpu.VMEM((FTAIL,), jnp.int32),
            ones_v=pltpu.VMEM((CS,), jnp.float32),
            ones_t=pltpu.VMEM((FTAIL,), jnp.float32),
            rows=[pltpu.VMEM((CS, D), jnp.float32) for _ in range(4)],
            rows_t=pltpu.VMEM((FTAIL, D), jnp.float32),
            isem=_SEM(4), gsem=_SEM(4), ssem=_SEM(4), dgsem=_SEM(4),
        ),
    )
    def k(feat_hbm, src_hbm, dst_hbm, z2_hbm, z1_hbm, *out_and_scratch,
          agg_sh, deg_sh, sidx, didx, sidx_t, didx_t, ones_v, ones_t,
          rows, rows_t, isem, gsem, ssem, dgsem):
        if with_deg:
            part_hbm, degp_hbm = out_and_scratch
        else:
            (part_hbm,) = out_and_scratch
            degp_hbm = None
        c = lax.axis_index("c")
        s = lax.axis_index("s")
        r0 = s * RPT

        pltpu.sync_copy(z2_hbm.at[pl.ds(r0, RPT)], agg_sh.at[pl.ds(r0, RPT)])
        if with_deg:
            pltpu.sync_copy(z1_hbm.at[pl.ds(r0, RPT)], deg_sh.at[pl.ds(r0, RPT)])
            _fill_ones(ones_v, CS)
            _fill_ones(ones_t, FTAIL)
        plsc.subcore_barrier()

        _agg_pipeline(feat_hbm, src_hbm, dst_hbm, agg_sh, deg_sh, sidx, didx,
                      rows, ones_v, isem, gsem, ssem, dgsem, sidx_t, didx_t,
                      rows_t, ones_t, c * EC + s * ET, with_deg)
        plsc.subcore_barrier()

        pltpu.sync_copy(agg_sh.at[pl.ds(r0, RPT)],
                        part_hbm.at[pl.ds(c * NP + r0, RPT)])
        if with_deg:
            pltpu.sync_copy(deg_sh.at[pl.ds(r0, RPT)],
                            degp_hbm.at[pl.ds(c * NP + r0, RPT)])

    return k(feat, src, dst, z2, z1)


def _sc_gather_out(y, src, dst):
    @functools.partial(
        pl.kernel,
        out_type=(
            jax.ShapeDtypeStruct((E, D), jnp.float32),
            jax.ShapeDtypeStruct((E, D), jnp.float32),
        ),
        mesh=_mesh,
        scratch_types=dict(
            y_sh=pltpu.VMEM_SHARED((N, D), jnp.float32),
            idx=[pltpu.VMEM((GCS,), jnp.int32) for _ in range(4)],
            rows=[pltpu.VMEM((GCS, D), jnp.float32) for _ in range(4)],
            isem=_SEM(4), gsem=_SEM(4), wsem=_SEM(4),
        ),
    )
    def k(y_hbm, src_hbm, dst_hbm, sf_hbm, df_hbm, *,
          y_sh, idx, rows, isem, gsem, wsem):
        c = lax.axis_index("c")
        s = lax.axis_index("s")
        base = (c * NS + s) * GT

        yr = s * YRT
        pltpu.sync_copy(y_hbm.at[pl.ds(yr, YRT)], y_sh.at[pl.ds(yr, YRT)])

        @pl.when(s == NS - 1)
        def _():
            pltpu.sync_copy(y_hbm.at[pl.ds(NS * YRT, N - NS * YRT)],
                            y_sh.at[pl.ds(NS * YRT, N - NS * YRT)])

        plsc.subcore_barrier()

        def off_of(jj):
            return pl.multiple_of(base + (jj // 2) * GCS, 8)

        def idx_issue(jj, b4, even):
            ref = src_hbm if even else dst_hbm
            pltpu.async_copy(ref.at[pl.ds(off_of(jj), GCS)], idx[b4], isem[b4])

        def idx_wait(b4):
            pltpu.make_async_copy(src_hbm.at[pl.ds(0, GCS)], idx[b4],
                                  isem[b4]).wait()

        def write_issue(jj, b4, even):
            out = sf_hbm if even else df_hbm
            pltpu.async_copy(rows[b4], out.at[pl.ds(off_of(jj), GCS)], wsem[b4])

        def write_wait(b4, even):
            out = sf_hbm if even else df_hbm
            pltpu.make_async_copy(rows[b4], out.at[pl.ds(0, GCS)],
                                  wsem[b4]).wait()

        def B(jj, b4, even):
            if not isinstance(jj, int) or jj >= 4:
                write_wait(b4, even)
            idx_wait(b4)
            pltpu.async_copy(y_sh.at[idx[b4]], rows[b4], gsem[b4])
            if not isinstance(jj, int) or jj >= 2:
                jp4 = (b4 + 2) % 4
                pltpu.make_async_copy(y_sh.at[idx[jp4]], rows[jp4],
                                      gsem[jp4]).wait()
                write_issue(jj - 2, jp4, even)
            if not isinstance(jj, int):
                idx_issue(jj + 2, (b4 + 2) % 4, even)
            elif jj + 2 < GCH:
                idx_issue(jj + 2, (b4 + 2) % 4, even)

        idx_issue(0, 0, True)
        idx_issue(1, 1, False)
        for jj in range(4):
            B(jj, jj % 4, jj % 2 == 0)

        @pl.loop(4, GCH - 6, step=4)
        def _(v):
            for u in range(4):
                B(v + u, u, u % 2 == 0)

        for jj in range(GCH - 6, GCH):
            B(jj, jj % 4, jj % 2 == 0)

        for jj in (GCH - 2, GCH - 1):
            b4, even = jj % 4, jj % 2 == 0
            pltpu.make_async_copy(y_sh.at[idx[b4]], rows[b4], gsem[b4]).wait()
            write_issue(jj, b4, even)
        for jj in range(GCH - 4, GCH):
            write_wait(jj % 4, jj % 2 == 0)

    return k(y, src, dst)


def _tc_dense(x, part, degp, W_s, W_n, b, relu):
    R = 1000
    part3 = part.reshape(2, NP, D)
    deg3 = degp.reshape(2, NP, 1)
    b2d = b.reshape(1, D)

    def body(x_ref, p0_ref, p1_ref, d_ref, ws_ref, wn_ref, b_ref, o_ref):
        dsum = d_ref[0] + d_ref[1]
        scale = 1.0 / jnp.maximum(dsum, 1.0)
        agg = (p0_ref[0] + p1_ref[0]) * scale
        acc = jnp.dot(x_ref[...], ws_ref[...], preferred_element_type=jnp.float32)
        acc = acc + jnp.dot(agg, wn_ref[...], preferred_element_type=jnp.float32)
        acc = acc + b_ref[...]
        if relu:
            acc = jnp.maximum(acc, 0.0)
        o_ref[...] = acc

    return pl.pallas_call(
        body,
        grid=(N // R,),
        in_specs=[
            pl.BlockSpec((R, D), lambda i: (i, 0)),
            pl.BlockSpec((1, R, D), lambda i: (0, i, 0)),
            pl.BlockSpec((1, R, D), lambda i: (1, i, 0)),
            pl.BlockSpec((2, R, 1), lambda i: (0, i, 0)),
            pl.BlockSpec((D, D), lambda i: (0, 0)),
            pl.BlockSpec((D, D), lambda i: (0, 0)),
            pl.BlockSpec((1, D), lambda i: (0, 0)),
        ],
        out_specs=pl.BlockSpec((R, D), lambda i: (i, 0)),
        out_shape=jax.ShapeDtypeStruct((N, D), jnp.float32),
    )(x, part3, part3, deg3, W_s, W_n, b2d)


def kernel(x, edge_index, W_self1, W_neigh1, b1, W_self2, W_neigh2, b2):
    src = edge_index[0].astype(jnp.int32)
    dst = edge_index[1].astype(jnp.int32)
    z2 = jnp.zeros((NP, D), jnp.float32)
    z1 = jnp.zeros((NP,), jnp.float32)

    part1, degp = _sc_agg(x, src, dst, z2, z1, with_deg=True)
    h = _tc_dense(x, part1, degp, W_self1, W_neigh1, b1, relu=True)
    (part2,) = _sc_agg(h, src, dst, z2, z1, with_deg=False)
    out2 = _tc_dense(h, part2, degp, W_self2, W_neigh2, b2, relu=False)
    src_feat, dst_feat = _sc_gather_out(out2, src, dst)
    return (src_feat, dst_feat)
